# baseline (speedup 1.0000x reference)
import functools

import jax
import jax.numpy as jnp
from jax import lax
from jax.experimental import pallas as pl
from jax.experimental.pallas import tpu as pltpu
from jax.experimental.pallas import tpu_sc as plsc

N = 10000
NP = 10240
D = 128
NC, NS = 2, 16
NW = NC * NS
RPT = NP // NS
K = 128
E = 320000
NCHUNK = -(-E // (NW * K))
EPW = NCHUNK * K
EPAD = EPW * NW

_MESH = dict(core_axis_name="c", subcore_axis_name="s",
             num_cores=NC, num_subcores=NS)


@functools.partial(
    pl.kernel,
    out_type=jax.ShapeDtypeStruct((NC, NP, D), jnp.float32),
    mesh=plsc.VectorSubcoreMesh(**_MESH),
    scratch_types=[
        pltpu.VMEM((2 * K,), jnp.int32),
        pltpu.VMEM((2 * K,), jnp.int32),
        pltpu.VMEM((K, D), jnp.float32),
        pltpu.VMEM((K, D), jnp.float32),
        pltpu.VMEM_SHARED((NP, D), jnp.float32),
        pltpu.SemaphoreType.DMA,
        pltpu.SemaphoreType.DMA,
        pltpu.SemaphoreType.DMA,
    ],
)
def _spmm(h_hbm, idx_hbm, zeros_hbm, out_hbm,
          idx0, idx1, rows0, rows1, accum, semg, sems0, sems1):
    c = lax.axis_index("c")
    s = lax.axis_index("s")
    wid = s * NC + c
    r0 = s * RPT
    pltpu.sync_copy(zeros_hbm, accum.at[pl.ds(r0, RPT)])
    plsc.subcore_barrier()
    base = wid * NCHUNK * 2 * K

    def gather(j, iv, rv):
        off = base + j * 2 * K
        pltpu.sync_copy(idx_hbm.at[pl.ds(off, 2 * K)], iv)
        pltpu.async_copy(h_hbm.at[iv.at[pl.ds(0, K)]], rv, semg).wait()

    def scat_start(rv, iv, sem):
        pltpu.async_copy(rv, accum.at[iv.at[pl.ds(K, K)]], sem, add=True)

    def scat_wait(rv, iv, sem):
        pltpu.make_async_copy(rv, accum.at[iv.at[pl.ds(K, K)]], sem).wait()

    gather(0, idx0, rows0)
    scat_start(rows0, idx0, sems0)

    def body(t, carry):
        j1 = 2 * t + 1
        gather(j1, idx1, rows1)
        scat_start(rows1, idx1, sems1)
        scat_wait(rows0, idx0, sems0)
        gather(j1 + 1, idx0, rows0)
        scat_start(rows0, idx0, sems0)
        scat_wait(rows1, idx1, sems1)
        return carry

    lax.fori_loop(0, (NCHUNK - 1) // 2, body, 0)
    scat_wait(rows0, idx0, sems0)
    plsc.subcore_barrier()
    pltpu.sync_copy(accum.at[pl.ds(r0, RPT)],
                    out_hbm.at[c, pl.ds(r0, RPT)])


@functools.partial(
    pl.kernel,
    out_type=jax.ShapeDtypeStruct((NC * NP,), jnp.float32),
    mesh=plsc.VectorSubcoreMesh(**_MESH),
    scratch_types=[
        pltpu.VMEM((K,), jnp.int32),
        pltpu.VMEM((K,), jnp.float32),
        pltpu.VMEM_SHARED((NP,), jnp.float32),
    ],
)
def _deg(dst_hbm, ones_hbm, zeros_hbm, out_hbm, dst_v, ones_v, accum):
    c = lax.axis_index("c")
    s = lax.axis_index("s")
    wid = s * NC + c
    r0 = s * RPT
    pltpu.sync_copy(zeros_hbm, accum.at[pl.ds(r0, RPT)])
    pltpu.sync_copy(ones_hbm, ones_v)
    plsc.subcore_barrier()
    base = wid * EPW

    def body(j, carry):
        off = base + j * K
        pltpu.sync_copy(dst_hbm.at[pl.ds(off, K)], dst_v)
        pltpu.sync_copy(ones_v, accum.at[dst_v], add=True)
        return carry

    lax.fori_loop(0, NCHUNK, body, 0)
    plsc.subcore_barrier()
    pltpu.sync_copy(accum.at[pl.ds(r0, RPT)],
                    out_hbm.at[pl.ds(c * NP + r0, RPT)])


BM = 1024


def _dinv_of(degp_ref):
    d = degp_ref[0, :] + degp_ref[1, :] + 1.0
    return lax.rsqrt(d)


def _mm(a, w):
    return lax.dot_general(a, w, (((1,), (0,)), ((), ())),
                           preferred_element_type=jnp.float32,
                           precision=lax.Precision.HIGHEST)


def _tc1_body(x_ref, w_ref, degp_ref, o_ref):
    dinv = _dinv_of(degp_ref)
    o_ref[...] = _mm(x_ref[...], w_ref[...]) * dinv[:, None]


def _tc_mid_body(p_ref, h_ref, degp_ref, b_ref, w_ref, o_ref):
    dinv = _dinv_of(degp_ref)
    tot = p_ref[0] + p_ref[1] + h_ref[...]
    xl = jnp.maximum(tot * dinv[:, None] + b_ref[...], 0.0)
    o_ref[...] = _mm(xl, w_ref[...]) * dinv[:, None]


def _tc_out_body(p_ref, h_ref, degp_ref, b_ref, o_ref):
    dinv = _dinv_of(degp_ref)
    tot = p_ref[0] + p_ref[1] + h_ref[...]
    o_ref[...] = tot * dinv[:, None] + b_ref[...]


def _degp_spec():
    return pl.BlockSpec((NC, BM), lambda i: (0, i))


def _tc1(x, w, degp):
    return pl.pallas_call(
        _tc1_body,
        grid=(NP // BM,),
        in_specs=[
            pl.BlockSpec((BM, D), lambda i: (i, 0)),
            pl.BlockSpec((D, D), lambda i: (0, 0)),
            _degp_spec(),
        ],
        out_specs=pl.BlockSpec((BM, D), lambda i: (i, 0)),
        out_shape=jax.ShapeDtypeStruct((NP, D), jnp.float32),
    )(x, w, degp)


def _tc_mid(p, h, degp, b, w):
    return pl.pallas_call(
        _tc_mid_body,
        grid=(NP // BM,),
        in_specs=[
            pl.BlockSpec((NC, BM, D), lambda i: (0, i, 0)),
            pl.BlockSpec((BM, D), lambda i: (i, 0)),
            _degp_spec(),
            pl.BlockSpec((1, D), lambda i: (0, 0)),
            pl.BlockSpec((D, D), lambda i: (0, 0)),
        ],
        out_specs=pl.BlockSpec((BM, D), lambda i: (i, 0)),
        out_shape=jax.ShapeDtypeStruct((NP, D), jnp.float32),
    )(p, h, degp, b, w)


def _tc_out(p, h, degp, b):
    return pl.pallas_call(
        _tc_out_body,
        grid=(NP // BM,),
        in_specs=[
            pl.BlockSpec((NC, BM, D), lambda i: (0, i, 0)),
            pl.BlockSpec((BM, D), lambda i: (i, 0)),
            _degp_spec(),
            pl.BlockSpec((1, D), lambda i: (0, 0)),
        ],
        out_specs=pl.BlockSpec((BM, D), lambda i: (i, 0)),
        out_shape=jax.ShapeDtypeStruct((NP, D), jnp.float32),
    )(p, h, degp, b)


def _padw(w):
    return jnp.pad(w, ((0, D - w.shape[0]), (0, D - w.shape[1])))


def kernel(x, edge_index, W1, b1, W2, b2, W_mu, b_mu, W_ls, b_ls):
    ei = edge_index.astype(jnp.int32)
    src = jnp.concatenate([ei[0], jnp.zeros((EPAD - E,), jnp.int32)])
    dst = jnp.concatenate([ei[1], jnp.full((EPAD - E,), N, jnp.int32)])
    idx = jnp.stack([src.reshape(NW * NCHUNK, K),
                     dst.reshape(NW * NCHUNK, K)], axis=1).reshape(-1)

    x_pad = jnp.pad(x, ((0, NP - N), (0, 0)))
    zrows = jnp.zeros((RPT, D), jnp.float32)
    z1 = jnp.zeros((RPT,), jnp.float32)
    ones1 = jnp.ones((K,), jnp.float32)

    degp = _deg(dst, ones1, z1).reshape(NC, NP)

    h1 = _tc1(x_pad, W1, degp)
    p1 = _spmm(h1, idx, zrows)

    h2 = _tc_mid(p1, h1, degp, b1[None, :], _padw(W2))
    p2 = _spmm(h2, idx, zrows)

    Wcat = jnp.concatenate([W_mu, W_ls], axis=1)
    bcat = jnp.concatenate([b_mu, b_ls])
    h3 = _tc_mid(p2, h2, degp, jnp.pad(b2, (0, D - b2.shape[0]))[None, :],
                 _padw(Wcat))
    p3 = _spmm(h3, idx, zrows)

    out = _tc_out(p3, h3, degp,
                  jnp.pad(bcat, (0, D - bcat.shape[0]))[None, :])
    mu = out[:N, :32]
    logstd = out[:N, 32:64]
    return (mu, logstd)


# speedup vs baseline: 1.0887x; 1.0887x over previous
import functools

import jax
import jax.numpy as jnp
from jax import lax
from jax.experimental import pallas as pl
from jax.experimental.pallas import tpu as pltpu
from jax.experimental.pallas import tpu_sc as plsc

N = 10000
NP = 10240
D = 128
NC, NS = 2, 16
NW = NC * NS
RPT = NP // NS
K = 128
E = 320000
NCHUNK = -(-E // (NW * K))
EPW = NCHUNK * K
EPAD = EPW * NW

_MESH = dict(core_axis_name="c", subcore_axis_name="s",
             num_cores=NC, num_subcores=NS)


@functools.partial(
    pl.kernel,
    out_type=jax.ShapeDtypeStruct((NC, NP, D), jnp.float32),
    mesh=plsc.VectorSubcoreMesh(**_MESH),
    scratch_types=[
        pltpu.VMEM((2 * K * 4,), jnp.int32),
        pltpu.VMEM((2 * K * 4,), jnp.int32),
        pltpu.VMEM((K, D), jnp.float32),
        pltpu.VMEM((K, D), jnp.float32),
        pltpu.VMEM_SHARED((NP, D), jnp.float32),
        pltpu.SemaphoreType.DMA,
        pltpu.SemaphoreType.DMA,
        pltpu.SemaphoreType.DMA,
    ],
)
def _spmm(h_hbm, idx_hbm, zeros_hbm, out_hbm,
          idxA, idxB, rows0, rows1, accum, semg, sems0, sems1):
    c = lax.axis_index("c")
    s = lax.axis_index("s")
    wid = s * NC + c
    r0 = s * RPT
    pltpu.sync_copy(zeros_hbm, accum.at[pl.ds(r0, RPT)])
    plsc.subcore_barrier()
    base = wid * NCHUNK * 2 * K
    rows = (rows0, rows1)
    sems = (sems0, sems1)

    def load(iv, j0):
        pltpu.sync_copy(idx_hbm.at[pl.ds(base + j0 * 2 * K, 8 * K)], iv)

    def step(iv, q, first=False):
        p = q % 2
        rv, sem = rows[p], sems[p]
        pltpu.async_copy(h_hbm.at[iv.at[pl.ds(q * 2 * K, K)]], rv, semg).wait()
        pltpu.async_copy(rv, accum.at[iv.at[pl.ds(q * 2 * K + K, K)]],
                         sem, add=True)
        if not first:
            pltpu.make_async_copy(rows[1 - p], accum.at[iv.at[pl.ds(K, K)]],
                                  sems[1 - p]).wait()

    load(idxA, 0)
    step(idxA, 0, first=True)
    for q in (1, 2, 3):
        step(idxA, q)

    def body(t, carry):
        j0 = 4 + 8 * t
        load(idxB, j0)
        for q in (0, 1, 2, 3):
            step(idxB, q)
        load(idxA, j0 + 4)
        for q in (0, 1, 2, 3):
            step(idxA, q)
        return carry

    lax.fori_loop(0, 9, body, 0)
    load(idxB, 76)
    for q in (0, 1, 2):
        step(idxB, q)
    pltpu.make_async_copy(rows0, accum.at[idxB.at[pl.ds(K, K)]],
                          sems0).wait()
    plsc.subcore_barrier()
    pltpu.sync_copy(accum.at[pl.ds(r0, RPT)],
                    out_hbm.at[c, pl.ds(r0, RPT)])


@functools.partial(
    pl.kernel,
    out_type=jax.ShapeDtypeStruct((NC * NP,), jnp.float32),
    mesh=plsc.VectorSubcoreMesh(**_MESH),
    scratch_types=[
        pltpu.VMEM((K,), jnp.int32),
        pltpu.VMEM((K,), jnp.float32),
        pltpu.VMEM_SHARED((NP,), jnp.float32),
    ],
)
def _deg(dst_hbm, ones_hbm, zeros_hbm, out_hbm, dst_v, ones_v, accum):
    c = lax.axis_index("c")
    s = lax.axis_index("s")
    wid = s * NC + c
    r0 = s * RPT
    pltpu.sync_copy(zeros_hbm, accum.at[pl.ds(r0, RPT)])
    pltpu.sync_copy(ones_hbm, ones_v)
    plsc.subcore_barrier()
    base = wid * EPW

    def body(j, carry):
        off = base + j * K
        pltpu.sync_copy(dst_hbm.at[pl.ds(off, K)], dst_v)
        pltpu.sync_copy(ones_v, accum.at[dst_v], add=True)
        return carry

    lax.fori_loop(0, NCHUNK, body, 0)
    plsc.subcore_barrier()
    pltpu.sync_copy(accum.at[pl.ds(r0, RPT)],
                    out_hbm.at[pl.ds(c * NP + r0, RPT)])


BM = 1024


def _dinv_of(degp_ref):
    d = degp_ref[0, :] + degp_ref[1, :] + 1.0
    return lax.rsqrt(d)


def _mm(a, w):
    return lax.dot_general(a, w, (((1,), (0,)), ((), ())),
                           preferred_element_type=jnp.float32,
                           precision=lax.Precision.HIGHEST)


def _tc1_body(x_ref, w_ref, degp_ref, o_ref):
    dinv = _dinv_of(degp_ref)
    o_ref[...] = _mm(x_ref[...], w_ref[...]) * dinv[:, None]


def _tc_mid_body(p_ref, h_ref, degp_ref, b_ref, w_ref, o_ref):
    dinv = _dinv_of(degp_ref)
    tot = p_ref[0] + p_ref[1] + h_ref[...]
    xl = jnp.maximum(tot * dinv[:, None] + b_ref[...], 0.0)
    o_ref[...] = _mm(xl, w_ref[...]) * dinv[:, None]


def _tc_out_body(p_ref, h_ref, degp_ref, b_ref, o_ref):
    dinv = _dinv_of(degp_ref)
    tot = p_ref[0] + p_ref[1] + h_ref[...]
    o_ref[...] = tot * dinv[:, None] + b_ref[...]


def _degp_spec():
    return pl.BlockSpec((NC, BM), lambda i: (0, i))


def _tc1(x, w, degp):
    return pl.pallas_call(
        _tc1_body,
        grid=(NP // BM,),
        in_specs=[
            pl.BlockSpec((BM, D), lambda i: (i, 0)),
            pl.BlockSpec((D, D), lambda i: (0, 0)),
            _degp_spec(),
        ],
        out_specs=pl.BlockSpec((BM, D), lambda i: (i, 0)),
        out_shape=jax.ShapeDtypeStruct((NP, D), jnp.float32),
    )(x, w, degp)


def _tc_mid(p, h, degp, b, w):
    return pl.pallas_call(
        _tc_mid_body,
        grid=(NP // BM,),
        in_specs=[
            pl.BlockSpec((NC, BM, D), lambda i: (0, i, 0)),
            pl.BlockSpec((BM, D), lambda i: (i, 0)),
            _degp_spec(),
            pl.BlockSpec((1, D), lambda i: (0, 0)),
            pl.BlockSpec((D, D), lambda i: (0, 0)),
        ],
        out_specs=pl.BlockSpec((BM, D), lambda i: (i, 0)),
        out_shape=jax.ShapeDtypeStruct((NP, D), jnp.float32),
    )(p, h, degp, b, w)


def _tc_out(p, h, degp, b):
    return pl.pallas_call(
        _tc_out_body,
        grid=(NP // BM,),
        in_specs=[
            pl.BlockSpec((NC, BM, D), lambda i: (0, i, 0)),
            pl.BlockSpec((BM, D), lambda i: (i, 0)),
            _degp_spec(),
            pl.BlockSpec((1, D), lambda i: (0, 0)),
        ],
        out_specs=pl.BlockSpec((BM, D), lambda i: (i, 0)),
        out_shape=jax.ShapeDtypeStruct((NP, D), jnp.float32),
    )(p, h, degp, b)


def _padw(w):
    return jnp.pad(w, ((0, D - w.shape[0]), (0, D - w.shape[1])))


def kernel(x, edge_index, W1, b1, W2, b2, W_mu, b_mu, W_ls, b_ls):
    ei = edge_index.astype(jnp.int32)
    src = jnp.concatenate([ei[0], jnp.zeros((EPAD - E,), jnp.int32)])
    dst = jnp.concatenate([ei[1], jnp.full((EPAD - E,), N, jnp.int32)])
    idx = jnp.concatenate(
        [jnp.stack([src.reshape(NW * NCHUNK, K),
                    dst.reshape(NW * NCHUNK, K)], axis=1).reshape(-1),
         jnp.zeros((2 * K,), jnp.int32)])

    x_pad = jnp.pad(x, ((0, NP - N), (0, 0)))
    zrows = jnp.zeros((RPT, D), jnp.float32)
    z1 = jnp.zeros((RPT,), jnp.float32)
    ones1 = jnp.ones((K,), jnp.float32)

    degp = _deg(dst, ones1, z1).reshape(NC, NP)

    h1 = _tc1(x_pad, W1, degp)
    p1 = _spmm(h1, idx, zrows)

    h2 = _tc_mid(p1, h1, degp, b1[None, :], _padw(W2))
    p2 = _spmm(h2, idx, zrows)

    Wcat = jnp.concatenate([W_mu, W_ls], axis=1)
    bcat = jnp.concatenate([b_mu, b_ls])
    h3 = _tc_mid(p2, h2, degp, jnp.pad(b2, (0, D - b2.shape[0]))[None, :],
                 _padw(Wcat))
    p3 = _spmm(h3, idx, zrows)

    out = _tc_out(p3, h3, degp,
                  jnp.pad(bcat, (0, D - bcat.shape[0]))[None, :])
    mu = out[:N, :32]
    logstd = out[:N, 32:64]
    return (mu, logstd)


# speedup vs baseline: 1.1009x; 1.0112x over previous
import functools

import jax
import jax.numpy as jnp
from jax import lax
from jax.experimental import pallas as pl
from jax.experimental.pallas import tpu as pltpu
from jax.experimental.pallas import tpu_sc as plsc

N = 10000
NP = 10240
D = 128
NC, NS = 2, 16
NW = NC * NS
RPT = NP // NS
K = 128
E = 320000
NCHUNK = -(-E // (NW * K))
EPW = NCHUNK * K
EPAD = EPW * NW

_MESH = dict(core_axis_name="c", subcore_axis_name="s",
             num_cores=NC, num_subcores=NS)


@functools.partial(
    pl.kernel,
    out_type=jax.ShapeDtypeStruct((NC, NP, D), jnp.float32),
    mesh=plsc.VectorSubcoreMesh(**_MESH),
    scratch_types=[
        pltpu.VMEM((2 * K * 8,), jnp.int32),
        pltpu.VMEM((2 * K * 8,), jnp.int32),
        pltpu.VMEM((K, D), jnp.float32),
        pltpu.VMEM((K, D), jnp.float32),
        pltpu.VMEM_SHARED((NP, D), jnp.float32),
        pltpu.SemaphoreType.DMA,
        pltpu.SemaphoreType.DMA,
        pltpu.SemaphoreType.DMA,
    ],
)
def _spmm(h_hbm, idx_hbm, zeros_hbm, out_hbm,
          idxA, idxB, rows0, rows1, accum, semg, sems0, sems1):
    c = lax.axis_index("c")
    s = lax.axis_index("s")
    wid = s * NC + c
    r0 = s * RPT
    pltpu.sync_copy(zeros_hbm, accum.at[pl.ds(r0, RPT)])
    plsc.subcore_barrier()
    base = wid * NCHUNK * 2 * K
    rows = (rows0, rows1)
    sems = (sems0, sems1)

    def load(iv, j0):
        pltpu.sync_copy(idx_hbm.at[pl.ds(base + j0 * 2 * K, 16 * K)], iv)

    def step(iv, q, first=False):
        p = q % 2
        rv, sem = rows[p], sems[p]
        pltpu.async_copy(h_hbm.at[iv.at[pl.ds(q * 2 * K, K)]], rv, semg).wait()
        pltpu.async_copy(rv, accum.at[iv.at[pl.ds(q * 2 * K + K, K)]],
                         sem, add=True)
        if not first:
            pltpu.make_async_copy(rows[1 - p], accum.at[iv.at[pl.ds(K, K)]],
                                  sems[1 - p]).wait()

    load(idxA, 0)
    step(idxA, 0, first=True)
    for q in range(1, 8):
        step(idxA, q)

    def body(t, carry):
        j0 = 8 + 16 * t
        load(idxB, j0)
        for q in range(8):
            step(idxB, q)
        load(idxA, j0 + 8)
        for q in range(8):
            step(idxA, q)
        return carry

    lax.fori_loop(0, 4, body, 0)
    load(idxB, 72)
    for q in range(7):
        step(idxB, q)
    pltpu.make_async_copy(rows0, accum.at[idxB.at[pl.ds(K, K)]],
                          sems0).wait()
    plsc.subcore_barrier()
    pltpu.sync_copy(accum.at[pl.ds(r0, RPT)],
                    out_hbm.at[c, pl.ds(r0, RPT)])


@functools.partial(
    pl.kernel,
    out_type=jax.ShapeDtypeStruct((NC * NP,), jnp.float32),
    mesh=plsc.VectorSubcoreMesh(**_MESH),
    scratch_types=[
        pltpu.VMEM((K,), jnp.int32),
        pltpu.VMEM((K,), jnp.float32),
        pltpu.VMEM_SHARED((NP,), jnp.float32),
    ],
)
def _deg(dst_hbm, ones_hbm, zeros_hbm, out_hbm, dst_v, ones_v, accum):
    c = lax.axis_index("c")
    s = lax.axis_index("s")
    wid = s * NC + c
    r0 = s * RPT
    pltpu.sync_copy(zeros_hbm, accum.at[pl.ds(r0, RPT)])
    pltpu.sync_copy(ones_hbm, ones_v)
    plsc.subcore_barrier()
    base = wid * EPW

    def body(j, carry):
        off = base + j * K
        pltpu.sync_copy(dst_hbm.at[pl.ds(off, K)], dst_v)
        pltpu.sync_copy(ones_v, accum.at[dst_v], add=True)
        return carry

    lax.fori_loop(0, NCHUNK, body, 0)
    plsc.subcore_barrier()
    pltpu.sync_copy(accum.at[pl.ds(r0, RPT)],
                    out_hbm.at[pl.ds(c * NP + r0, RPT)])


BM = 1024


def _dinv_of(degp_ref):
    d = degp_ref[0, :] + degp_ref[1, :] + 1.0
    return lax.rsqrt(d)


def _mm(a, w):
    return lax.dot_general(a, w, (((1,), (0,)), ((), ())),
                           preferred_element_type=jnp.float32,
                           precision=lax.Precision.HIGHEST)


def _tc1_body(x_ref, w_ref, degp_ref, o_ref):
    dinv = _dinv_of(degp_ref)
    o_ref[...] = _mm(x_ref[...], w_ref[...]) * dinv[:, None]


def _tc_mid_body(p_ref, h_ref, degp_ref, b_ref, w_ref, o_ref):
    dinv = _dinv_of(degp_ref)
    tot = p_ref[0] + p_ref[1] + h_ref[...]
    xl = jnp.maximum(tot * dinv[:, None] + b_ref[...], 0.0)
    o_ref[...] = _mm(xl, w_ref[...]) * dinv[:, None]


def _tc_out_body(p_ref, h_ref, degp_ref, b_ref, o_ref):
    dinv = _dinv_of(degp_ref)
    tot = p_ref[0] + p_ref[1] + h_ref[...]
    o_ref[...] = tot * dinv[:, None] + b_ref[...]


def _degp_spec():
    return pl.BlockSpec((NC, BM), lambda i: (0, i))


def _tc1(x, w, degp):
    return pl.pallas_call(
        _tc1_body,
        grid=(NP // BM,),
        in_specs=[
            pl.BlockSpec((BM, D), lambda i: (i, 0)),
            pl.BlockSpec((D, D), lambda i: (0, 0)),
            _degp_spec(),
        ],
        out_specs=pl.BlockSpec((BM, D), lambda i: (i, 0)),
        out_shape=jax.ShapeDtypeStruct((NP, D), jnp.float32),
    )(x, w, degp)


def _tc_mid(p, h, degp, b, w):
    return pl.pallas_call(
        _tc_mid_body,
        grid=(NP // BM,),
        in_specs=[
            pl.BlockSpec((NC, BM, D), lambda i: (0, i, 0)),
            pl.BlockSpec((BM, D), lambda i: (i, 0)),
            _degp_spec(),
            pl.BlockSpec((1, D), lambda i: (0, 0)),
            pl.BlockSpec((D, D), lambda i: (0, 0)),
        ],
        out_specs=pl.BlockSpec((BM, D), lambda i: (i, 0)),
        out_shape=jax.ShapeDtypeStruct((NP, D), jnp.float32),
    )(p, h, degp, b, w)


def _tc_out(p, h, degp, b):
    return pl.pallas_call(
        _tc_out_body,
        grid=(NP // BM,),
        in_specs=[
            pl.BlockSpec((NC, BM, D), lambda i: (0, i, 0)),
            pl.BlockSpec((BM, D), lambda i: (i, 0)),
            _degp_spec(),
            pl.BlockSpec((1, D), lambda i: (0, 0)),
        ],
        out_specs=pl.BlockSpec((BM, D), lambda i: (i, 0)),
        out_shape=jax.ShapeDtypeStruct((NP, D), jnp.float32),
    )(p, h, degp, b)


def _padw(w):
    return jnp.pad(w, ((0, D - w.shape[0]), (0, D - w.shape[1])))


def kernel(x, edge_index, W1, b1, W2, b2, W_mu, b_mu, W_ls, b_ls):
    ei = edge_index.astype(jnp.int32)
    src = jnp.concatenate([ei[0], jnp.zeros((EPAD - E,), jnp.int32)])
    dst = jnp.concatenate([ei[1], jnp.full((EPAD - E,), N, jnp.int32)])
    idx = jnp.concatenate(
        [jnp.stack([src.reshape(NW * NCHUNK, K),
                    dst.reshape(NW * NCHUNK, K)], axis=1).reshape(-1),
         jnp.zeros((2 * K,), jnp.int32)])

    x_pad = jnp.pad(x, ((0, NP - N), (0, 0)))
    zrows = jnp.zeros((RPT, D), jnp.float32)
    z1 = jnp.zeros((RPT,), jnp.float32)
    ones1 = jnp.ones((K,), jnp.float32)

    degp = _deg(dst, ones1, z1).reshape(NC, NP)

    h1 = _tc1(x_pad, W1, degp)
    p1 = _spmm(h1, idx, zrows)

    h2 = _tc_mid(p1, h1, degp, b1[None, :], _padw(W2))
    p2 = _spmm(h2, idx, zrows)

    Wcat = jnp.concatenate([W_mu, W_ls], axis=1)
    bcat = jnp.concatenate([b_mu, b_ls])
    h3 = _tc_mid(p2, h2, degp, jnp.pad(b2, (0, D - b2.shape[0]))[None, :],
                 _padw(Wcat))
    p3 = _spmm(h3, idx, zrows)

    out = _tc_out(p3, h3, degp,
                  jnp.pad(bcat, (0, D - bcat.shape[0]))[None, :])
    mu = out[:N, :32]
    logstd = out[:N, 32:64]
    return (mu, logstd)
